# Initial kernel scaffold; baseline (speedup 1.0000x reference)
import jax
import jax.numpy as jnp
from jax.experimental import pallas as pl


def kernel(score, ans_idx):
    raise NotImplementedError("write your pallas kernel here")


# speedup vs baseline: 13.3537x; 13.3537x over previous
import functools

import jax
import jax.numpy as jnp
from jax import lax
from jax.experimental import pallas as pl
from jax.experimental.pallas import tpu as pltpu
from jax.experimental.pallas import tpu_sc as plsc

BATCH = 128
ROW = 8192
LANES = 16
NUM_CORES = 2
NUM_SUBCORES = 16
NUM_WORKERS = NUM_CORES * NUM_SUBCORES
ROWS_PER_W = BATCH // NUM_WORKERS
NUM_CHUNKS = ROW // LANES


def _sc_body(score_hbm, ans_hbm, out_hbm, s_v, a_v, st_v):
    wid = lax.axis_index("s") * NUM_CORES + lax.axis_index("c")
    base = wid * ROWS_PER_W
    pltpu.sync_copy(score_hbm.at[pl.ds(base, ROWS_PER_W)], s_v)
    pltpu.sync_copy(ans_hbm.at[pl.ds(base, ROWS_PER_W)], a_v)

    iota = lax.iota(jnp.int32, LANES)
    partial = jnp.zeros((LANES,), jnp.float32)
    for r in range(ROWS_PER_W):
        def chunk(c, carry, r=r):
            vmax, vval, vidx = carry
            off = c * LANES
            s = s_v[r, pl.ds(off, LANES)]
            a = a_v[r, pl.ds(off, LANES)]
            cidx = off + iota
            pred = s > vmax
            return (
                jnp.where(pred, s, vmax),
                jnp.where(pred, a, vval),
                jnp.where(pred, cidx, vidx),
            )

        init = (
            jnp.full((LANES,), -jnp.inf, jnp.float32),
            jnp.zeros((LANES,), jnp.float32),
            jnp.zeros((LANES,), jnp.int32),
        )
        vmax, vval, vidx = lax.fori_loop(0, NUM_CHUNKS, chunk, init)
        m = jnp.max(vmax)
        cand = jnp.where(vmax == m, vidx, jnp.int32(1 << 30))
        imin = jnp.min(cand)
        hit = (vidx == imin) & (vmax == m)
        partial = partial + jnp.where(hit, vval, jnp.float32(0.0))

    st_v[...] = partial
    pltpu.sync_copy(st_v, out_hbm.at[wid])


@jax.jit
def _sc_partials(score, ans_idx):
    mesh = plsc.VectorSubcoreMesh(core_axis_name="c", subcore_axis_name="s")
    return pl.kernel(
        _sc_body,
        out_type=jax.ShapeDtypeStruct((NUM_WORKERS, LANES), jnp.float32),
        mesh=mesh,
        scratch_types=[
            pltpu.VMEM((ROWS_PER_W, ROW), jnp.float32),
            pltpu.VMEM((ROWS_PER_W, ROW), jnp.float32),
            pltpu.VMEM((LANES,), jnp.float32),
        ],
        compiler_params=pltpu.CompilerParams(needs_layout_passes=False),
    )(score, ans_idx)


def _reduce_body(p_ref, o_ref):
    o_ref[0, 0] = jnp.sum(p_ref[...]) * (100.0 / BATCH)


@jax.jit
def _tc_reduce(partials):
    return pl.pallas_call(
        _reduce_body,
        out_shape=jax.ShapeDtypeStruct((1, 1), jnp.float32),
        in_specs=[pl.BlockSpec(memory_space=pltpu.VMEM)],
        out_specs=pl.BlockSpec(memory_space=pltpu.SMEM),
    )(partials)


def kernel(score, ans_idx):
    partials = _sc_partials(score, ans_idx)
    acc = _tc_reduce(partials.reshape(4, BATCH))
    return acc[0, 0]


# speedup vs baseline: 15.1737x; 1.1363x over previous
import functools

import jax
import jax.numpy as jnp
from jax import lax
from jax.experimental import pallas as pl
from jax.experimental.pallas import tpu as pltpu
from jax.experimental.pallas import tpu_sc as plsc

BATCH = 128
ROW = 8192
LANES = 16
NUM_CORES = 2
NUM_SUBCORES = 16
NUM_WORKERS = NUM_CORES * NUM_SUBCORES
ROWS_PER_W = BATCH // NUM_WORKERS
NUM_CHUNKS = ROW // LANES
UNROLL = 8


def _sc_body(score_hbm, ans_flat_hbm, out_hbm, s_v, g_v, st_v, sem):
    wid = lax.axis_index("s") * NUM_CORES + lax.axis_index("c")
    base = wid * ROWS_PER_W
    pltpu.sync_copy(score_hbm.at[pl.ds(base, ROWS_PER_W)], s_v)

    iota = lax.iota(jnp.int32, LANES)
    ones = jnp.ones((LANES,), jnp.int32)
    idxv = jnp.zeros((LANES,), jnp.int32)
    for r in range(ROWS_PER_W):
        def chunk(c, carry, r=r):
            vmax, vchunk, vcnt = carry
            for u in range(UNROLL):
                s = s_v[r, pl.ds(c * (UNROLL * LANES) + u * LANES, LANES)]
                pred = s > vmax
                vmax = jnp.where(pred, s, vmax)
                vchunk = jnp.where(pred, vcnt, vchunk)
                vcnt = vcnt + ones
            return vmax, vchunk, vcnt

        init = (
            jnp.full((LANES,), -jnp.inf, jnp.float32),
            jnp.zeros((LANES,), jnp.int32),
            jnp.zeros((LANES,), jnp.int32),
        )
        vmax, vchunk, _ = lax.fori_loop(0, NUM_CHUNKS // UNROLL, chunk, init)
        vidx = vchunk * LANES + iota
        m = jnp.max(vmax)
        cand = jnp.where(vmax == m, vidx, jnp.int32(1 << 30))
        imin = jnp.min(cand)
        flat = (base + r) * ROW + imin
        idxv = jnp.where(iota == r, flat, idxv)

    pltpu.async_copy(ans_flat_hbm.at[idxv], g_v, sem).wait()
    g = g_v[...]
    st_v[...] = jnp.where(iota < ROWS_PER_W, g, jnp.float32(0.0))
    pltpu.sync_copy(st_v, out_hbm.at[wid])


@jax.jit
def _sc_partials(score, ans_flat):
    mesh = plsc.VectorSubcoreMesh(core_axis_name="c", subcore_axis_name="s")
    return pl.kernel(
        _sc_body,
        out_type=jax.ShapeDtypeStruct((NUM_WORKERS, LANES), jnp.float32),
        mesh=mesh,
        scratch_types=[
            pltpu.VMEM((ROWS_PER_W, ROW), jnp.float32),
            pltpu.VMEM((LANES,), jnp.float32),
            pltpu.VMEM((LANES,), jnp.float32),
            pltpu.SemaphoreType.DMA,
        ],
        compiler_params=pltpu.CompilerParams(needs_layout_passes=False),
    )(score, ans_flat)


def _reduce_body(p_ref, o_ref):
    o_ref[0, 0] = jnp.sum(p_ref[...]) * (100.0 / BATCH)


@jax.jit
def _tc_reduce(partials):
    return pl.pallas_call(
        _reduce_body,
        out_shape=jax.ShapeDtypeStruct((1, 1), jnp.float32),
        in_specs=[pl.BlockSpec(memory_space=pltpu.VMEM)],
        out_specs=pl.BlockSpec(memory_space=pltpu.SMEM),
    )(partials)


def kernel(score, ans_idx):
    partials = _sc_partials(score, ans_idx.reshape(-1))
    acc = _tc_reduce(partials)
    return acc[0, 0]
